# baseline (speedup 1.0000x reference)
import functools

import jax
import jax.numpy as jnp
from jax import lax
from jax.experimental import pallas as pl
from jax.experimental.pallas import tpu as pltpu
from jax.experimental.pallas import tpu_sc as plsc

VOCAB = 1000
EMBED = 64
BATCH = 16384
HIST = 50
HIST_PAD = 56
LANE_PAD = 128

_INFO = plsc.get_sparse_core_info()
_NC = _INFO.num_cores
_NS = _INFO.num_subcores
_NW = _NC * _NS

_BATCH_PER_W = BATCH // _NW
_NB = 8
_NCHUNK = _BATCH_PER_W // _NB
_NPAIR = _NCHUNK // 2


def _make_kernel():
  mesh = plsc.VectorSubcoreMesh(core_axis_name="c", subcore_axis_name="s")

  @functools.partial(
      pl.kernel,
      mesh=mesh,
      compiler_params=pltpu.CompilerParams(use_tc_tiling_on_sc=False),
      out_type=jax.ShapeDtypeStruct((BATCH, HIST_PAD, LANE_PAD), jnp.float32),
      scratch_types=[
          pltpu.VMEM((_BATCH_PER_W, LANE_PAD), jnp.int32),
          pltpu.VMEM((2, _NB, HIST_PAD, EMBED), jnp.float32),
          pltpu.SemaphoreType.DMA,
          pltpu.SemaphoreType.DMA,
          pltpu.SemaphoreType.DMA,
      ],
  )
  def gather_kernel(idx_hbm, table_hbm, out_hbm, idx_all, rows, gsem, s0, s1):
    wid = lax.axis_index("s") * _NC + lax.axis_index("c")
    base = wid * _BATCH_PER_W
    ssems = (s0, s1)

    def run_gather(c, b):
      copies = [
          pltpu.async_copy(
              table_hbm.at[idx_all.at[c * _NB + j, pl.ds(0, HIST_PAD)]],
              rows.at[b].at[j],
              gsem,
          )
          for j in range(_NB)
      ]
      for cp in copies:
        cp.wait()

    def fire_store(c, b):
      pltpu.async_copy(
          rows.at[b],
          out_hbm.at[pl.ds(base + c * _NB, _NB), pl.ds(0, HIST_PAD), pl.ds(0, EMBED)],
          ssems[b],
      )

    def wait_store(b):
      pltpu.make_async_copy(
          rows.at[b],
          out_hbm.at[pl.ds(0, _NB), pl.ds(0, HIST_PAD), pl.ds(0, EMBED)],
          ssems[b],
      ).wait()

    pltpu.sync_copy(idx_hbm.at[wid], idx_all)

    def pair_body(p, carry):
      for b in range(2):
        c = 2 * p + b

        @pl.when(c >= 2)
        def _():
          wait_store(b)

        run_gather(c, b)
        fire_store(c, b)
      return carry

    lax.fori_loop(0, _NPAIR, pair_body, 0)
    wait_store(0)
    wait_store(1)

  return gather_kernel


_GATHER = _make_kernel()


def kernel(word_idx_list, W):
  idx = word_idx_list.astype(jnp.int32)
  idx = jnp.pad(idx, ((0, 0), (0, LANE_PAD - HIST)))
  out = _GATHER(idx.reshape(_NW, _BATCH_PER_W, LANE_PAD), W)
  return out[:, :HIST, :EMBED]
